# baseline (speedup 1.0000x reference)
import functools

import jax
import jax.numpy as jnp
from jax import lax
from jax.experimental import pallas as pl
from jax.experimental.pallas import tpu as pltpu
from jax.experimental.pallas import tpu_sc as plsc

N_NODES = 50000
IN_FEATS = 512
NUM_GRAPHS = 256
NW = 32
PER_W = 1568
CHUNK = 64
L = 16


def _set_idx(idx_buf, cur):
    lanes = lax.iota(jnp.int32, L)
    plsc.store_scatter(idx_buf, [jnp.zeros((L,), jnp.int32)],
                       jnp.full((L,), cur, jnp.int32), mask=lanes == 0)


def _zero_acc(acc_buf):
    z = jnp.zeros((L,), jnp.float32)
    for j in range(IN_FEATS // L):
        acc_buf[0, pl.ds(j * L, L)] = z


def _dot16(row_buf, w_vmem, b_s, g16):
    row_ids = g16 + lax.iota(jnp.int32, L)

    def jgbody(jg, acc):
        wv = w_vmem[pl.ds(jg * L, L)]
        for l in range(L):
            col = plsc.load_gather(
                row_buf, [row_ids, jnp.full((L,), jg * L + l, jnp.int32)])
            acc = acc + col * wv[l]
        return acc

    acc = lax.fori_loop(0, IN_FEATS // L, jgbody, jnp.zeros((L,), jnp.float32))
    return acc + b_s


def _process_chunk(pos, n_groups, cur_seg, feats, ids, aw_out,
                   row_buf, ids_buf, aw_buf, w_buf, acc_buf, idx_buf,
                   w_vmem, b_s, hg_acc):
    nrows = n_groups * L
    pltpu.sync_copy(feats.at[pl.ds(pos, nrows)], row_buf.at[pl.ds(0, nrows)])
    pltpu.sync_copy(ids.at[pl.ds(pos, nrows)], ids_buf.at[pl.ds(0, nrows)])
    for g in range(n_groups):
        aw16 = _dot16(row_buf, w_vmem, b_s, g * L)
        w16 = 1.0 / (1.0 + jnp.exp(-aw16))
        aw_buf[pl.ds(g * L, L)] = aw16
        w_buf[pl.ds(g * L, L)] = w16
    pltpu.sync_copy(aw_buf.at[pl.ds(0, nrows)], aw_out.at[pl.ds(pos, nrows)])

    def rbody(r, cur):
        s_r = ids_buf[pl.ds(r, L)][0]
        w_r = w_buf[pl.ds(r, L)][0]

        @pl.when(s_r != cur)
        def _():
            _set_idx(idx_buf, cur)
            pltpu.sync_copy(acc_buf, hg_acc.at[idx_buf], add=True)
            _zero_acc(acc_buf)

        for j in range(IN_FEATS // L):
            acc_buf[0, pl.ds(j * L, L)] += w_r * row_buf[r, pl.ds(j * L, L)]
        return s_r

    return lax.fori_loop(0, nrows, rbody, cur_seg)


def _sc_body(feats, ids, w_hbm, b_hbm, zeros_hbm, aw_out, hg_part,
             row_buf, ids_buf, aw_buf, w_buf, acc_buf, idx_buf,
             w_vmem, b_vmem, hg_acc):
    cid = lax.axis_index("c")
    sid = lax.axis_index("s")
    wid = cid * 16 + sid

    pltpu.sync_copy(w_hbm, w_vmem)
    pltpu.sync_copy(b_hbm, b_vmem)
    b_s = b_vmem[...][0]

    @pl.when(sid == 0)
    def _():
        pltpu.sync_copy(zeros_hbm, hg_acc)

    plsc.subcore_barrier()
    _zero_acc(acc_buf)

    start = wid * PER_W
    count = jnp.minimum(PER_W, N_NODES - start)
    nfull = count // CHUNK
    ntail = (count - nfull * CHUNK) // L

    args = (feats, ids, aw_out, row_buf, ids_buf, aw_buf, w_buf, acc_buf,
            idx_buf, w_vmem, b_s, hg_acc)

    def full_body(k, cur):
        return _process_chunk(start + k * CHUNK, CHUNK // L, cur, *args)

    cur_seg = lax.fori_loop(0, nfull, full_body, jnp.int32(NUM_GRAPHS))

    def tail_body(k, cur):
        return _process_chunk(start + nfull * CHUNK + k * L, 1, cur, *args)

    cur_seg = lax.fori_loop(0, ntail, tail_body, cur_seg)

    _set_idx(idx_buf, cur_seg)
    pltpu.sync_copy(acc_buf, hg_acc.at[idx_buf], add=True)

    plsc.subcore_barrier()
    off = cid * NUM_GRAPHS + sid * L
    pltpu.sync_copy(hg_acc.at[pl.ds(sid * L, L)], hg_part.at[pl.ds(off, L)])


@jax.jit
def _sc_call(feats, ids, w1, b8, zeros):
    mesh = plsc.VectorSubcoreMesh(core_axis_name="c", subcore_axis_name="s",
                                  num_cores=2, num_subcores=16)
    return pl.kernel(
        _sc_body,
        out_type=[
            jax.ShapeDtypeStruct((N_NODES,), jnp.float32),
            jax.ShapeDtypeStruct((2 * NUM_GRAPHS, IN_FEATS), jnp.float32),
        ],
        mesh=mesh,
        compiler_params=pltpu.CompilerParams(use_tc_tiling_on_sc=False, needs_layout_passes=False),
        scratch_types=[
            pltpu.VMEM((CHUNK, IN_FEATS), jnp.float32),
            pltpu.VMEM((CHUNK + L,), jnp.int32),
            pltpu.VMEM((CHUNK + L,), jnp.float32),
            pltpu.VMEM((CHUNK + L,), jnp.float32),
            pltpu.VMEM((1, IN_FEATS), jnp.float32),
            pltpu.VMEM((1,), jnp.int32),
            pltpu.VMEM((IN_FEATS,), jnp.float32),
            pltpu.VMEM((L,), jnp.float32),
            pltpu.VMEM_SHARED((NUM_GRAPHS + 1, IN_FEATS), jnp.float32),
        ],
    )(feats, ids, w1, b8, zeros)


def _merge_body(p_ref, out_ref):
    out_ref[...] = p_ref[0] + p_ref[1]


def kernel(feats, segment_ids, W, b):
    ids32 = segment_ids.astype(jnp.int32)
    w1 = W.reshape(IN_FEATS)
    b8 = jnp.concatenate([b.astype(jnp.float32), jnp.zeros((L - 1,), jnp.float32)])
    zeros = jnp.zeros((NUM_GRAPHS + 1, IN_FEATS), jnp.float32)
    aw_flat, hg_part = _sc_call(feats, ids32, w1, b8, zeros)
    hg = pl.pallas_call(
        _merge_body,
        out_shape=jax.ShapeDtypeStruct((NUM_GRAPHS, IN_FEATS), jnp.float32),
    )(hg_part.reshape(2, NUM_GRAPHS, IN_FEATS))
    return (hg, aw_flat.reshape(N_NODES, 1))
